# baseline (speedup 1.0000x reference)
import functools

import jax
import jax.numpy as jnp
from jax import lax
from jax.experimental import pallas as pl
from jax.experimental.pallas import tpu as pltpu
from jax.experimental.pallas import tpu_sc as plsc

_D = 512
_NSENT = 4096
_W = 50
_WP = 56
_NC, _NS = 2, 16
_NW = _NC * _NS
_SPW = _NSENT // _NW
_S = 2
_NCH = _SPW // _S
_NBUF = 2


def _make_gather():
  mesh = plsc.VectorSubcoreMesh(core_axis_name="c", subcore_axis_name="s")
  scratch = [pltpu.VMEM((_NCH, 1, _S * _WP), jnp.int32)]
  scratch += [pltpu.VMEM((_S * _WP, _D), jnp.float32) for _ in range(_NBUF)]
  scratch += [pltpu.SemaphoreType.DMA for _ in range(2 * _NBUF)]

  @functools.partial(
      pl.kernel,
      mesh=mesh,
      out_type=jax.ShapeDtypeStruct((_NSENT, _WP, _D), jnp.float32),
      scratch_types=scratch,
      compiler_params=pltpu.CompilerParams(use_tc_tiling_on_sc=True),
  )
  def gather_kernel(idx_hbm, table_hbm, out_hbm, idx_v, *rest):
    bufs = rest[:_NBUF]
    in_sems = rest[_NBUF:2 * _NBUF]
    out_sems = rest[2 * _NBUF:]
    wid = lax.axis_index("s") * _NC + lax.axis_index("c")
    sent0 = wid * _SPW

    pltpu.sync_copy(idx_hbm.at[pl.ds(wid * _NCH, _NCH)], idx_v)

    def start_gather(c, b):
      pltpu.async_copy(table_hbm.at[idx_v.at[c, 0]], bufs[b], in_sems[b])

    def wait_gather(b):
      pltpu.make_async_copy(
          table_hbm.at[idx_v.at[0, 0]], bufs[b], in_sems[b]).wait()

    def start_scatter(c, b):
      for i in range(_S):
        pltpu.async_copy(
            bufs[b].at[pl.ds(i * _WP, _WP)],
            out_hbm.at[sent0 + c * _S + i], out_sems[b])

    def wait_scatter(b):
      for i in range(_S):
        pltpu.make_async_copy(
            bufs[b].at[pl.ds(i * _WP, _WP)],
            out_hbm.at[sent0 + i], out_sems[b]).wait()

    for b in range(_NBUF):
      start_gather(b, b)

    def body(o, carry):
      for b in range(_NBUF):
        c = o * _NBUF + b
        wait_gather(b)
        start_scatter(c, b)
        wait_scatter(b)
        start_gather(c + _NBUF, b)
      return carry

    lax.fori_loop(0, _NCH // _NBUF - 1, body, 0)

    for b in range(_NBUF):
      wait_gather(b)
      start_scatter(_NCH - _NBUF + b, b)
    for b in range(_NBUF):
      wait_scatter(b)

  return gather_kernel


_gather = _make_gather()


def kernel(x, table):
  xp = jnp.pad(x.astype(jnp.int32), ((0, 0), (0, _WP - _W)))
  idx = xp.reshape(_NW * _NCH, 1, _S * _WP)
  return _gather(idx, table)[:, :_W, :]


# speedup vs baseline: 2.1300x; 2.1300x over previous
import functools

import jax
import jax.numpy as jnp
from jax import lax
from jax.experimental import pallas as pl
from jax.experimental.pallas import tpu as pltpu
from jax.experimental.pallas import tpu_sc as plsc

_D = 512
_B = 4096 * 50
_NC, _NS = 2, 16
_NW = _NC * _NS
_RPW = _B // _NW
_CH = 80
_NCH = _RPW // _CH
_NBUF = 2


def _make_gather():
  mesh = plsc.VectorSubcoreMesh(core_axis_name="c", subcore_axis_name="s")
  scratch = [pltpu.VMEM((_NCH, 1, _CH), jnp.int32)]
  scratch += [pltpu.VMEM((_CH, _D), jnp.float32) for _ in range(_NBUF)]
  scratch += [pltpu.SemaphoreType.DMA for _ in range(2 * _NBUF)]

  @functools.partial(
      pl.kernel,
      mesh=mesh,
      out_type=jax.ShapeDtypeStruct((_B, _D), jnp.float32),
      scratch_types=scratch,
      compiler_params=pltpu.CompilerParams(use_tc_tiling_on_sc=True),
  )
  def gather_kernel(idx_hbm, table_hbm, out_hbm, idx_v, *rest):
    bufs = rest[:_NBUF]
    in_sems = rest[_NBUF:2 * _NBUF]
    out_sems = rest[2 * _NBUF:]
    wid = lax.axis_index("s") * _NC + lax.axis_index("c")
    row0 = wid * _RPW

    pltpu.sync_copy(idx_hbm.at[pl.ds(wid * _NCH, _NCH)], idx_v)

    def start_gather(c, b):
      pltpu.async_copy(table_hbm.at[idx_v.at[c, 0]], bufs[b], in_sems[b])

    def wait_gather(b):
      pltpu.make_async_copy(
          table_hbm.at[idx_v.at[0, 0]], bufs[b], in_sems[b]).wait()

    def start_scatter(c, b):
      pltpu.async_copy(
          bufs[b], out_hbm.at[pl.ds(row0 + c * _CH, _CH)], out_sems[b])

    def wait_scatter(b):
      pltpu.make_async_copy(
          bufs[b], out_hbm.at[pl.ds(row0, _CH)], out_sems[b]).wait()

    for b in range(_NBUF):
      start_gather(b, b)

    def body(o, carry):
      for b in range(_NBUF):
        c = o * _NBUF + b
        wait_gather(b)
        start_scatter(c, b)
        wait_scatter(b)
        start_gather(c + _NBUF, b)
      return carry

    lax.fori_loop(0, _NCH // _NBUF - 1, body, 0)

    for b in range(_NBUF):
      wait_gather(b)
      start_scatter(_NCH - _NBUF + b, b)
    for b in range(_NBUF):
      wait_scatter(b)

  return gather_kernel


_gather = _make_gather()


def kernel(x, table):
  idx = x.astype(jnp.int32).reshape(_NW * _NCH, 1, _CH)
  return _gather(idx, table).reshape(4096, 50, _D)
